# Initial kernel scaffold; baseline (speedup 1.0000x reference)
import jax
import jax.numpy as jnp
from jax.experimental import pallas as pl


def kernel(pos, x, features, batch, W1, b1, W2, b2, W3, b3, W4, b4, Wf1, bf1, Wa, ba, Wb, bb, Wc, bc):
    raise NotImplementedError("write your pallas kernel here")


# speedup vs baseline: 2.3002x; 2.3002x over previous
import functools

import jax
import jax.numpy as jnp
from jax.experimental import pallas as pl

N = 4096
B = 8
KNN = 20
BIG = 1e30


def _lrelu(v):
    return jnp.where(v >= 0, v, 0.2 * v)


def _edge_body(xb_ref, x_ref, xT_ref, bcol_ref, brow_ref, w_ref, b_ref, o_ref):
    xb = xb_ref[...]
    xT = xT_ref[...]
    d2b = jnp.sum(xb * xb, axis=1, keepdims=True)
    d2r = jnp.sum(xT * xT, axis=0, keepdims=True)
    xx = jnp.dot(xb, xT, preferred_element_type=jnp.float32)
    D = d2b + d2r - 2.0 * xx
    mask = bcol_ref[...] != brow_ref[...]
    D = jnp.where(mask, BIG, D)
    iota = jax.lax.broadcasted_iota(jnp.int32, D.shape, 1)
    x = x_ref[...]
    wbf = w_ref[...].astype(jnp.bfloat16)
    R = D.shape[0]
    out = wbf.shape[1]
    M0 = jnp.full((R, out), -BIG, jnp.float32)

    def step(_, carry):
        D, M = carry
        m = jnp.min(D, axis=1, keepdims=True)
        am = jnp.min(jnp.where(D == m, iota, N), axis=1, keepdims=True)
        oh = iota == am
        xj = jnp.dot(oh.astype(jnp.float32), x, preferred_element_type=jnp.float32,
                     precision=jax.lax.Precision.HIGHEST)
        msg = jnp.concatenate([xb, xj - xb], axis=1).astype(jnp.bfloat16)
        h = jnp.dot(msg, wbf, preferred_element_type=jnp.float32)
        return jnp.where(oh, BIG, D), jnp.maximum(M, h)

    _, M = jax.lax.fori_loop(0, KNN, step, (D, M0))
    o_ref[...] = _lrelu(M + b_ref[...])


def _edge_conv(x, xT, bcol, brow, W, b, R=256):
    d = x.shape[1]
    out = W.shape[1]
    return pl.pallas_call(
        _edge_body,
        grid=(N // R,),
        in_specs=[
            pl.BlockSpec((R, d), lambda i: (i, 0)),
            pl.BlockSpec((N, d), lambda i: (0, 0)),
            pl.BlockSpec((d, N), lambda i: (0, 0)),
            pl.BlockSpec((R, 1), lambda i: (i, 0)),
            pl.BlockSpec((1, N), lambda i: (0, 0)),
            pl.BlockSpec((2 * d, out), lambda i: (0, 0)),
            pl.BlockSpec((1, out), lambda i: (0, 0)),
        ],
        out_specs=pl.BlockSpec((R, out), lambda i: (i, 0)),
        out_shape=jax.ShapeDtypeStruct((N, out), jnp.float32),
    )(x, x, xT, bcol, brow, W, b.reshape(1, out))


def _head_body(x1_ref, x2_ref, x3_ref, x4_ref, wf_ref, bf_ref, bcol_ref, g_ref):
    h = jnp.concatenate(
        [x1_ref[...], x2_ref[...], x3_ref[...], x4_ref[...]], axis=1
    )
    h = _lrelu(jnp.dot(h, wf_ref[...], preferred_element_type=jnp.float32) + bf_ref[...])

    @pl.when(pl.program_id(0) == 0)
    def _():
        g_ref[...] = jnp.full(g_ref.shape, -jnp.inf, jnp.float32)

    bcol = bcol_ref[...]
    for seg in range(B):
        v = jnp.max(jnp.where(bcol == seg, h, -jnp.inf), axis=0, keepdims=True)
        g_ref[seg:seg + 1, :] = jnp.maximum(g_ref[seg:seg + 1, :], v)


def _head(x1, x2, x3, x4, Wf1, bf1, bcol, S=512):
    F = Wf1.shape[1]
    return pl.pallas_call(
        _head_body,
        grid=(N // S,),
        in_specs=[
            pl.BlockSpec((S, x1.shape[1]), lambda i: (i, 0)),
            pl.BlockSpec((S, x2.shape[1]), lambda i: (i, 0)),
            pl.BlockSpec((S, x3.shape[1]), lambda i: (i, 0)),
            pl.BlockSpec((S, x4.shape[1]), lambda i: (i, 0)),
            pl.BlockSpec(Wf1.shape, lambda i: (0, 0)),
            pl.BlockSpec((1, F), lambda i: (0, 0)),
            pl.BlockSpec((S, 1), lambda i: (i, 0)),
        ],
        out_specs=pl.BlockSpec((B, F), lambda i: (0, 0)),
        out_shape=jax.ShapeDtypeStruct((B, F), jnp.float32),
    )(x1, x2, x3, x4, Wf1, bf1.reshape(1, F), bcol)


def _head2_body(g_ref, wa_ref, ba_ref, wb_ref, bb_ref, wc_ref, bc_ref, o_ref):
    g = jnp.maximum(jnp.dot(g_ref[...], wa_ref[...], preferred_element_type=jnp.float32) + ba_ref[...], 0.0)
    g = jnp.maximum(jnp.dot(g, wb_ref[...], preferred_element_type=jnp.float32) + bb_ref[...], 0.0)
    z = jnp.dot(g, wc_ref[...], preferred_element_type=jnp.float32) + bc_ref[...]
    zmax = jnp.max(z, axis=1, keepdims=True)
    s = jnp.sum(jnp.exp(z - zmax), axis=1, keepdims=True)
    o_ref[...] = z - zmax - jnp.log(s)


def _head2(g, Wa, ba, Wb, bb, Wc, bc):
    out = Wc.shape[1]
    return pl.pallas_call(
        _head2_body,
        out_shape=jax.ShapeDtypeStruct((B, out), jnp.float32),
    )(g, Wa, ba.reshape(1, -1), Wb, bb.reshape(1, -1), Wc, bc.reshape(1, -1))


def kernel(pos, x, features, batch, W1, b1, W2, b2, W3, b3, W4, b4,
           Wf1, bf1, Wa, ba, Wb, bb, Wc, bc):
    bcol = batch.astype(jnp.int32).reshape(N, 1)
    brow = bcol.reshape(1, N)
    h = jnp.concatenate([pos, x, features], axis=1)
    outs = []
    for (W, b) in ((W1, b1), (W2, b2), (W3, b3), (W4, b4)):
        h = _edge_conv(h, h.T, bcol, brow, W, b)
        outs.append(h)
    g = _head(outs[0], outs[1], outs[2], outs[3], Wf1, bf1, bcol)
    return _head2(g, Wa, ba, Wb, bb, Wc, bc)


# speedup vs baseline: 2.3008x; 1.0002x over previous
import functools

import jax
import jax.numpy as jnp
from jax.experimental import pallas as pl
from jax.experimental.pallas import tpu as pltpu

N = 4096
B = 8
KNN = 20
BIG = 1e30


def _lrelu(v):
    return jnp.where(v >= 0, v, 0.2 * v)


def _edge_body(xb_ref, x_ref, xT_ref, bcol_ref, brow_ref, w_ref, b_ref, o_ref):
    xb = xb_ref[...]
    xT = xT_ref[...]
    d2b = jnp.sum(xb * xb, axis=1, keepdims=True)
    d2r = jnp.sum(xT * xT, axis=0, keepdims=True)
    xx = jnp.dot(xb, xT, preferred_element_type=jnp.float32)
    D = d2b + d2r - 2.0 * xx
    mask = bcol_ref[...] != brow_ref[...]
    D = jnp.where(mask, BIG, D)
    iota = jax.lax.broadcasted_iota(jnp.int32, D.shape, 1)
    x = x_ref[...]
    wbf = w_ref[...].astype(jnp.bfloat16)
    R = D.shape[0]
    out = wbf.shape[1]
    M0 = jnp.full((R, out), -BIG, jnp.float32)

    def step(_, carry):
        D, M = carry
        m = jnp.min(D, axis=1, keepdims=True)
        am = jnp.min(jnp.where(D == m, iota, N), axis=1, keepdims=True)
        oh = iota == am
        xj = jnp.dot(oh.astype(jnp.float32), x, preferred_element_type=jnp.float32,
                     precision=jax.lax.Precision.HIGHEST)
        msg = jnp.concatenate([xb, xj - xb], axis=1).astype(jnp.bfloat16)
        h = jnp.dot(msg, wbf, preferred_element_type=jnp.float32)
        return jnp.where(oh, BIG, D), jnp.maximum(M, h)

    _, M = jax.lax.fori_loop(0, KNN, step, (D, M0))
    o_ref[...] = _lrelu(M + b_ref[...])


def _edge_conv(x, xT, bcol, brow, W, b, R=256):
    d = x.shape[1]
    out = W.shape[1]
    return pl.pallas_call(
        _edge_body,
        grid=(N // R,),
        in_specs=[
            pl.BlockSpec((R, d), lambda i: (i, 0)),
            pl.BlockSpec((N, d), lambda i: (0, 0)),
            pl.BlockSpec((d, N), lambda i: (0, 0)),
            pl.BlockSpec((R, 1), lambda i: (i, 0)),
            pl.BlockSpec((1, N), lambda i: (0, 0)),
            pl.BlockSpec((2 * d, out), lambda i: (0, 0)),
            pl.BlockSpec((1, out), lambda i: (0, 0)),
        ],
        out_specs=pl.BlockSpec((R, out), lambda i: (i, 0)),
        out_shape=jax.ShapeDtypeStruct((N, out), jnp.float32),
        compiler_params=pltpu.CompilerParams(
            dimension_semantics=("parallel",)),
    )(x, x, xT, bcol, brow, W, b.reshape(1, out))


def _head_body(x1_ref, x2_ref, x3_ref, x4_ref, wf_ref, bf_ref, bcol_ref, g_ref):
    h = jnp.concatenate(
        [x1_ref[...], x2_ref[...], x3_ref[...], x4_ref[...]], axis=1
    )
    h = _lrelu(jnp.dot(h, wf_ref[...], preferred_element_type=jnp.float32) + bf_ref[...])

    @pl.when(pl.program_id(0) == 0)
    def _():
        g_ref[...] = jnp.full(g_ref.shape, -jnp.inf, jnp.float32)

    bcol = bcol_ref[...]
    for seg in range(B):
        v = jnp.max(jnp.where(bcol == seg, h, -jnp.inf), axis=0, keepdims=True)
        g_ref[seg:seg + 1, :] = jnp.maximum(g_ref[seg:seg + 1, :], v)


def _head(x1, x2, x3, x4, Wf1, bf1, bcol, S=512):
    F = Wf1.shape[1]
    return pl.pallas_call(
        _head_body,
        grid=(N // S,),
        in_specs=[
            pl.BlockSpec((S, x1.shape[1]), lambda i: (i, 0)),
            pl.BlockSpec((S, x2.shape[1]), lambda i: (i, 0)),
            pl.BlockSpec((S, x3.shape[1]), lambda i: (i, 0)),
            pl.BlockSpec((S, x4.shape[1]), lambda i: (i, 0)),
            pl.BlockSpec(Wf1.shape, lambda i: (0, 0)),
            pl.BlockSpec((1, F), lambda i: (0, 0)),
            pl.BlockSpec((S, 1), lambda i: (i, 0)),
        ],
        out_specs=pl.BlockSpec((B, F), lambda i: (0, 0)),
        out_shape=jax.ShapeDtypeStruct((B, F), jnp.float32),
    )(x1, x2, x3, x4, Wf1, bf1.reshape(1, F), bcol)


def _head2_body(g_ref, wa_ref, ba_ref, wb_ref, bb_ref, wc_ref, bc_ref, o_ref):
    g = jnp.maximum(jnp.dot(g_ref[...], wa_ref[...], preferred_element_type=jnp.float32) + ba_ref[...], 0.0)
    g = jnp.maximum(jnp.dot(g, wb_ref[...], preferred_element_type=jnp.float32) + bb_ref[...], 0.0)
    z = jnp.dot(g, wc_ref[...], preferred_element_type=jnp.float32) + bc_ref[...]
    zmax = jnp.max(z, axis=1, keepdims=True)
    s = jnp.sum(jnp.exp(z - zmax), axis=1, keepdims=True)
    o_ref[...] = z - zmax - jnp.log(s)


def _head2(g, Wa, ba, Wb, bb, Wc, bc):
    out = Wc.shape[1]
    return pl.pallas_call(
        _head2_body,
        out_shape=jax.ShapeDtypeStruct((B, out), jnp.float32),
    )(g, Wa, ba.reshape(1, -1), Wb, bb.reshape(1, -1), Wc, bc.reshape(1, -1))


def kernel(pos, x, features, batch, W1, b1, W2, b2, W3, b3, W4, b4,
           Wf1, bf1, Wa, ba, Wb, bb, Wc, bc):
    bcol = batch.astype(jnp.int32).reshape(N, 1)
    brow = bcol.reshape(1, N)
    h = jnp.concatenate([pos, x, features], axis=1)
    outs = []
    for (W, b) in ((W1, b1), (W2, b2), (W3, b3), (W4, b4)):
        h = _edge_conv(h, h.T, bcol, brow, W, b)
        outs.append(h)
    g = _head(outs[0], outs[1], outs[2], outs[3], Wf1, bf1, bcol)
    return _head2(g, Wa, ba, Wb, bb, Wc, bc)


# speedup vs baseline: 5.7722x; 2.5088x over previous
import functools

import jax
import jax.numpy as jnp
from jax import lax
from jax.experimental import pallas as pl
from jax.experimental.pallas import tpu as pltpu
from jax.experimental.pallas import tpu_sc as plsc

N = 4096
B = 8
KNN = 20
BIG = 1e30


def _lrelu(v):
    return jnp.where(v >= 0, v, 0.2 * v)


KP = 24


def _knn_body(xb_ref, xT_ref, bcol_ref, brow_ref, idx_ref):
    xb = xb_ref[...]
    xT = xT_ref[...]
    d2b = jnp.sum(xb * xb, axis=1, keepdims=True)
    d2r = jnp.sum(xT * xT, axis=0, keepdims=True)
    xx = jnp.dot(xb, xT, preferred_element_type=jnp.float32)
    D = d2b + d2r - 2.0 * xx
    mask = bcol_ref[...] != brow_ref[...]
    D = jnp.where(mask, BIG, D)
    iota = jax.lax.broadcasted_iota(jnp.int32, D.shape, 1)
    R = D.shape[0]
    base = pl.program_id(0) * R
    kiota = jax.lax.broadcasted_iota(jnp.int32, (R, KP), 1)
    riota = base + jax.lax.broadcasted_iota(jnp.int32, (R, KP), 0)
    IDX0 = riota

    def step(t, carry):
        D, IDX = carry
        m = jnp.min(D, axis=1, keepdims=True)
        am = jnp.min(jnp.where(D == m, iota, N), axis=1, keepdims=True)
        IDX = jnp.where(kiota == t, am, IDX)
        return jnp.where(iota == am, BIG, D), IDX

    _, IDX = jax.lax.fori_loop(0, KNN, step, (D, IDX0))
    idx_ref[...] = IDX


def _knn(x, xT, bcol, brow, R=256):
    d = x.shape[1]
    return pl.pallas_call(
        _knn_body,
        grid=(N // R,),
        in_specs=[
            pl.BlockSpec((R, d), lambda i: (i, 0)),
            pl.BlockSpec((d, N), lambda i: (0, 0)),
            pl.BlockSpec((R, 1), lambda i: (i, 0)),
            pl.BlockSpec((1, N), lambda i: (0, 0)),
        ],
        out_specs=pl.BlockSpec((R, KP), lambda i: (i, 0)),
        out_shape=jax.ShapeDtypeStruct((N, KP), jnp.int32),
        compiler_params=pltpu.CompilerParams(
            dimension_semantics=("parallel",)),
    )(x, xT, bcol, brow)


_NC = 2
_NS = 16
_NW = _NC * _NS
_C = N // _NW


def _sc_gather(x, idxT):
    d = x.shape[1]
    mesh = plsc.VectorSubcoreMesh(core_axis_name="c", subcore_axis_name="s")

    @functools.partial(
        pl.kernel, mesh=mesh,
        out_type=jax.ShapeDtypeStruct((KNN, N, d), jnp.float32),
        scratch_types=[
            pltpu.VMEM((_C,), jnp.int32),
            pltpu.VMEM((_C,), jnp.int32),
            pltpu.VMEM((_C, d), jnp.float32),
            pltpu.VMEM((_C, d), jnp.float32),
            pltpu.SemaphoreType.DMA,
            pltpu.SemaphoreType.DMA,
        ],
    )
    def k(x_hbm, idxT_hbm, xg_hbm, idx_v0, idx_v1, buf0, buf1, sem0, sem1):
        wid = lax.axis_index("s") * _NC + lax.axis_index("c")
        base = wid * _C
        idxvs = (idx_v0, idx_v1)
        bufs = (buf0, buf1)
        sems = (sem0, sem1)
        cps = []
        for k_ in range(KNN):
            pltpu.sync_copy(idxT_hbm.at[k_, pl.ds(base, _C)], idxvs[k_ % 2])
            cp = pltpu.async_copy(x_hbm.at[idxvs[k_ % 2]], bufs[k_ % 2], sems[k_ % 2])
            if k_ > 0:
                cps[k_ - 1].wait()
                pltpu.sync_copy(bufs[(k_ - 1) % 2],
                                xg_hbm.at[k_ - 1, pl.ds(base, _C), :])
            cps.append(cp)
        cps[KNN - 1].wait()
        pltpu.sync_copy(bufs[(KNN - 1) % 2],
                        xg_hbm.at[KNN - 1, pl.ds(base, _C), :])

    return k(x, idxT)


def _edge_body(d, xb_ref, xg_ref, w_ref, b_ref, o_ref):
    xb = xb_ref[...][:, :d]
    wbf = w_ref[...].astype(jnp.bfloat16)
    R = xb.shape[0]
    out = wbf.shape[1]
    M = jnp.full((R, out), -BIG, jnp.float32)
    for k_ in range(KNN):
        xj = xg_ref[k_][:, :d]
        msg = jnp.concatenate([xb, xj - xb], axis=1).astype(jnp.bfloat16)
        h = jnp.dot(msg, wbf, preferred_element_type=jnp.float32)
        M = jnp.maximum(M, h)
    o_ref[...] = _lrelu(M + b_ref[...])


def _edge_mlp(x, xg, W, b, d, R=512):
    dp = x.shape[1]
    out = W.shape[1]
    return pl.pallas_call(
        functools.partial(_edge_body, d),
        grid=(N // R,),
        in_specs=[
            pl.BlockSpec((R, dp), lambda i: (i, 0)),
            pl.BlockSpec((KNN, R, dp), lambda i: (0, i, 0)),
            pl.BlockSpec((2 * d, out), lambda i: (0, 0)),
            pl.BlockSpec((1, out), lambda i: (0, 0)),
        ],
        out_specs=pl.BlockSpec((R, out), lambda i: (i, 0)),
        out_shape=jax.ShapeDtypeStruct((N, out), jnp.float32),
        compiler_params=pltpu.CompilerParams(
            dimension_semantics=("parallel",)),
    )(x, xg, W, b.reshape(1, out))


def _edge_conv(x, bcol, brow, W, b):
    d = x.shape[1]
    dp = -(-d // 128) * 128
    xp = jnp.pad(x, ((0, 0), (0, dp - d))) if dp != d else x
    idx = _knn(x, x.T, bcol, brow)
    idxT = idx.T[:KNN]
    xg = _sc_gather(xp, idxT)
    return _edge_mlp(xp, xg, W, b, d)


def _head_body(x1_ref, x2_ref, x3_ref, x4_ref, wf_ref, bf_ref, bcol_ref, g_ref):
    h = jnp.concatenate(
        [x1_ref[...], x2_ref[...], x3_ref[...], x4_ref[...]], axis=1
    )
    h = _lrelu(jnp.dot(h, wf_ref[...], preferred_element_type=jnp.float32) + bf_ref[...])

    @pl.when(pl.program_id(0) == 0)
    def _():
        g_ref[...] = jnp.full(g_ref.shape, -jnp.inf, jnp.float32)

    bcol = bcol_ref[...]
    for seg in range(B):
        v = jnp.max(jnp.where(bcol == seg, h, -jnp.inf), axis=0, keepdims=True)
        g_ref[seg:seg + 1, :] = jnp.maximum(g_ref[seg:seg + 1, :], v)


def _head(x1, x2, x3, x4, Wf1, bf1, bcol, S=512):
    F = Wf1.shape[1]
    return pl.pallas_call(
        _head_body,
        grid=(N // S,),
        in_specs=[
            pl.BlockSpec((S, x1.shape[1]), lambda i: (i, 0)),
            pl.BlockSpec((S, x2.shape[1]), lambda i: (i, 0)),
            pl.BlockSpec((S, x3.shape[1]), lambda i: (i, 0)),
            pl.BlockSpec((S, x4.shape[1]), lambda i: (i, 0)),
            pl.BlockSpec(Wf1.shape, lambda i: (0, 0)),
            pl.BlockSpec((1, F), lambda i: (0, 0)),
            pl.BlockSpec((S, 1), lambda i: (i, 0)),
        ],
        out_specs=pl.BlockSpec((B, F), lambda i: (0, 0)),
        out_shape=jax.ShapeDtypeStruct((B, F), jnp.float32),
    )(x1, x2, x3, x4, Wf1, bf1.reshape(1, F), bcol)


def _head2_body(g_ref, wa_ref, ba_ref, wb_ref, bb_ref, wc_ref, bc_ref, o_ref):
    g = jnp.maximum(jnp.dot(g_ref[...], wa_ref[...], preferred_element_type=jnp.float32) + ba_ref[...], 0.0)
    g = jnp.maximum(jnp.dot(g, wb_ref[...], preferred_element_type=jnp.float32) + bb_ref[...], 0.0)
    z = jnp.dot(g, wc_ref[...], preferred_element_type=jnp.float32) + bc_ref[...]
    zmax = jnp.max(z, axis=1, keepdims=True)
    s = jnp.sum(jnp.exp(z - zmax), axis=1, keepdims=True)
    o_ref[...] = z - zmax - jnp.log(s)


def _head2(g, Wa, ba, Wb, bb, Wc, bc):
    out = Wc.shape[1]
    return pl.pallas_call(
        _head2_body,
        out_shape=jax.ShapeDtypeStruct((B, out), jnp.float32),
    )(g, Wa, ba.reshape(1, -1), Wb, bb.reshape(1, -1), Wc, bc.reshape(1, -1))


def kernel(pos, x, features, batch, W1, b1, W2, b2, W3, b3, W4, b4,
           Wf1, bf1, Wa, ba, Wb, bb, Wc, bc):
    bcol = batch.astype(jnp.int32).reshape(N, 1)
    brow = bcol.reshape(1, N)
    h = jnp.concatenate([pos, x, features], axis=1)
    outs = []
    for (W, b) in ((W1, b1), (W2, b2), (W3, b3), (W4, b4)):
        h = _edge_conv(h, bcol, brow, W, b)
        outs.append(h)
    g = _head(outs[0], outs[1], outs[2], outs[3], Wf1, bf1, bcol)
    return _head2(g, Wa, ba, Wb, bb, Wc, bc)


# speedup vs baseline: 9.9930x; 1.7312x over previous
import functools

import jax
import jax.numpy as jnp
from jax import lax
from jax.experimental import pallas as pl
from jax.experimental.pallas import tpu as pltpu
from jax.experimental.pallas import tpu_sc as plsc

N = 4096
B = 8
KNN = 20
BIG = 1e30


def _lrelu(v):
    return jnp.where(v >= 0, v, 0.2 * v)


KP = 24


WIN = 2048


def _knn_body(win_ref, xb_ref, xT_ref, bcol_ref, brow_ref, idx_ref):
    i = pl.program_id(0)
    start = pl.multiple_of(win_ref[i, 0], 128)
    wide = win_ref[i, 1]
    xb = xb_ref[...]
    d2b = jnp.sum(xb * xb, axis=1, keepdims=True)
    bcol = bcol_ref[...]
    R = xb.shape[0]
    base = pl.program_id(0) * R

    def run(colstart, W_):
        xT = xT_ref[:, pl.ds(colstart, W_)]
        d2r = jnp.sum(xT * xT, axis=0, keepdims=True)
        xx = jnp.dot(xb, xT, preferred_element_type=jnp.float32)
        D = d2b + d2r - 2.0 * xx
        mask = bcol != brow_ref[:, pl.ds(colstart, W_)]
        D = jnp.where(mask, BIG, D)
        iota = colstart + jax.lax.broadcasted_iota(jnp.int32, (R, W_), 1)
        kiota = jax.lax.broadcasted_iota(jnp.int32, (R, KP), 1)
        riota = base + jax.lax.broadcasted_iota(jnp.int32, (R, KP), 0)

        def step(t, carry):
            D, IDX = carry
            m = jnp.min(D, axis=1, keepdims=True)
            am = jnp.min(jnp.where(D == m, iota, N), axis=1, keepdims=True)
            IDX = jnp.where(kiota == t, am, IDX)
            return jnp.where(iota == am, BIG, D), IDX

        _, IDX = jax.lax.fori_loop(0, KNN, step, (D, riota))
        idx_ref[...] = IDX

    @pl.when(wide == 0)
    def _():
        run(start, WIN)

    @pl.when(wide != 0)
    def _():
        run(0, N)


def _knn(x, xT, win, bcol, brow, R=256):
    d = x.shape[1]
    return pl.pallas_call(
        _knn_body,
        grid=(N // R,),
        in_specs=[
            pl.BlockSpec(memory_space=pltpu.SMEM),
            pl.BlockSpec((R, d), lambda i: (i, 0)),
            pl.BlockSpec((d, N), lambda i: (0, 0)),
            pl.BlockSpec((R, 1), lambda i: (i, 0)),
            pl.BlockSpec((1, N), lambda i: (0, 0)),
        ],
        out_specs=pl.BlockSpec((R, KP), lambda i: (i, 0)),
        out_shape=jax.ShapeDtypeStruct((N, KP), jnp.int32),
        compiler_params=pltpu.CompilerParams(
            dimension_semantics=("parallel",)),
    )(win, x, xT, bcol, brow)


def _windows(batch, R=256):
    firsts = batch[::R]
    lasts = batch[R - 1::R]
    seg_lo = jnp.searchsorted(batch, firsts, side="left").astype(jnp.int32)
    seg_hi = jnp.searchsorted(batch, lasts, side="right").astype(jnp.int32)
    sa = jnp.minimum(seg_lo, N - WIN) & ~jnp.int32(127)
    wide = (seg_hi > sa + WIN).astype(jnp.int32)
    return jnp.stack([sa, wide], axis=1)


_NC = 2
_NS = 16
_NW = _NC * _NS
_C = N // _NW


def _sc_gather(x, idxT):
    d = x.shape[1]
    mesh = plsc.VectorSubcoreMesh(core_axis_name="c", subcore_axis_name="s")

    @functools.partial(
        pl.kernel, mesh=mesh,
        out_type=jax.ShapeDtypeStruct((KNN, N, d), jnp.float32),
        scratch_types=[
            pltpu.VMEM((_C,), jnp.int32),
            pltpu.VMEM((_C,), jnp.int32),
            pltpu.VMEM((_C, d), jnp.float32),
            pltpu.VMEM((_C, d), jnp.float32),
            pltpu.SemaphoreType.DMA,
            pltpu.SemaphoreType.DMA,
        ],
    )
    def k(x_hbm, idxT_hbm, xg_hbm, idx_v0, idx_v1, buf0, buf1, sem0, sem1):
        wid = lax.axis_index("s") * _NC + lax.axis_index("c")
        base = wid * _C
        idxvs = (idx_v0, idx_v1)
        bufs = (buf0, buf1)
        sems = (sem0, sem1)
        cps = []
        for k_ in range(KNN):
            pltpu.sync_copy(idxT_hbm.at[k_, pl.ds(base, _C)], idxvs[k_ % 2])
            cp = pltpu.async_copy(x_hbm.at[idxvs[k_ % 2]], bufs[k_ % 2], sems[k_ % 2])
            if k_ > 0:
                cps[k_ - 1].wait()
                pltpu.sync_copy(bufs[(k_ - 1) % 2],
                                xg_hbm.at[k_ - 1, pl.ds(base, _C), :])
            cps.append(cp)
        cps[KNN - 1].wait()
        pltpu.sync_copy(bufs[(KNN - 1) % 2],
                        xg_hbm.at[KNN - 1, pl.ds(base, _C), :])

    return k(x, idxT)


def _edge_body(d, xb_ref, xg_ref, w_ref, b_ref, o_ref):
    xb = xb_ref[...][:, :d]
    wbf = w_ref[...].astype(jnp.bfloat16)
    R = xb.shape[0]
    out = wbf.shape[1]
    M = jnp.full((R, out), -BIG, jnp.float32)
    for k_ in range(KNN):
        xj = xg_ref[k_][:, :d]
        msg = jnp.concatenate([xb, xj - xb], axis=1).astype(jnp.bfloat16)
        h = jnp.dot(msg, wbf, preferred_element_type=jnp.float32)
        M = jnp.maximum(M, h)
    o_ref[...] = _lrelu(M + b_ref[...])


def _edge_mlp(x, xg, W, b, d, R=512):
    dp = x.shape[1]
    out = W.shape[1]
    return pl.pallas_call(
        functools.partial(_edge_body, d),
        grid=(N // R,),
        in_specs=[
            pl.BlockSpec((R, dp), lambda i: (i, 0)),
            pl.BlockSpec((KNN, R, dp), lambda i: (0, i, 0)),
            pl.BlockSpec((2 * d, out), lambda i: (0, 0)),
            pl.BlockSpec((1, out), lambda i: (0, 0)),
        ],
        out_specs=pl.BlockSpec((R, out), lambda i: (i, 0)),
        out_shape=jax.ShapeDtypeStruct((N, out), jnp.float32),
        compiler_params=pltpu.CompilerParams(
            dimension_semantics=("parallel",)),
    )(x, xg, W, b.reshape(1, out))


def _edge_conv(x, win, bcol, brow, W, b):
    d = x.shape[1]
    dp = -(-d // 128) * 128
    xp = jnp.pad(x, ((0, 0), (0, dp - d))) if dp != d else x
    idx = _knn(x, x.T, win, bcol, brow)
    idxT = idx.T[:KNN]
    xg = _sc_gather(xp, idxT)
    return _edge_mlp(xp, xg, W, b, d)


def _head_body(x1_ref, x2_ref, x3_ref, x4_ref, wf_ref, bf_ref, bcol_ref, g_ref):
    h = jnp.concatenate(
        [x1_ref[...], x2_ref[...], x3_ref[...], x4_ref[...]], axis=1
    )
    h = _lrelu(jnp.dot(h, wf_ref[...], preferred_element_type=jnp.float32) + bf_ref[...])

    @pl.when(pl.program_id(0) == 0)
    def _():
        g_ref[...] = jnp.full(g_ref.shape, -jnp.inf, jnp.float32)

    bcol = bcol_ref[...]
    for seg in range(B):
        v = jnp.max(jnp.where(bcol == seg, h, -jnp.inf), axis=0, keepdims=True)
        g_ref[seg:seg + 1, :] = jnp.maximum(g_ref[seg:seg + 1, :], v)


def _head(x1, x2, x3, x4, Wf1, bf1, bcol, S=512):
    F = Wf1.shape[1]
    return pl.pallas_call(
        _head_body,
        grid=(N // S,),
        in_specs=[
            pl.BlockSpec((S, x1.shape[1]), lambda i: (i, 0)),
            pl.BlockSpec((S, x2.shape[1]), lambda i: (i, 0)),
            pl.BlockSpec((S, x3.shape[1]), lambda i: (i, 0)),
            pl.BlockSpec((S, x4.shape[1]), lambda i: (i, 0)),
            pl.BlockSpec(Wf1.shape, lambda i: (0, 0)),
            pl.BlockSpec((1, F), lambda i: (0, 0)),
            pl.BlockSpec((S, 1), lambda i: (i, 0)),
        ],
        out_specs=pl.BlockSpec((B, F), lambda i: (0, 0)),
        out_shape=jax.ShapeDtypeStruct((B, F), jnp.float32),
    )(x1, x2, x3, x4, Wf1, bf1.reshape(1, F), bcol)


def _head2_body(g_ref, wa_ref, ba_ref, wb_ref, bb_ref, wc_ref, bc_ref, o_ref):
    g = jnp.maximum(jnp.dot(g_ref[...], wa_ref[...], preferred_element_type=jnp.float32) + ba_ref[...], 0.0)
    g = jnp.maximum(jnp.dot(g, wb_ref[...], preferred_element_type=jnp.float32) + bb_ref[...], 0.0)
    z = jnp.dot(g, wc_ref[...], preferred_element_type=jnp.float32) + bc_ref[...]
    zmax = jnp.max(z, axis=1, keepdims=True)
    s = jnp.sum(jnp.exp(z - zmax), axis=1, keepdims=True)
    o_ref[...] = z - zmax - jnp.log(s)


def _head2(g, Wa, ba, Wb, bb, Wc, bc):
    out = Wc.shape[1]
    return pl.pallas_call(
        _head2_body,
        out_shape=jax.ShapeDtypeStruct((B, out), jnp.float32),
    )(g, Wa, ba.reshape(1, -1), Wb, bb.reshape(1, -1), Wc, bc.reshape(1, -1))


def kernel(pos, x, features, batch, W1, b1, W2, b2, W3, b3, W4, b4,
           Wf1, bf1, Wa, ba, Wb, bb, Wc, bc):
    bcol = batch.astype(jnp.int32).reshape(N, 1)
    brow = bcol.reshape(1, N)
    win = _windows(batch.astype(jnp.int32))
    h = jnp.concatenate([pos, x, features], axis=1)
    outs = []
    for (W, b) in ((W1, b1), (W2, b2), (W3, b3), (W4, b4)):
        h = _edge_conv(h, win, bcol, brow, W, b)
        outs.append(h)
    g = _head(outs[0], outs[1], outs[2], outs[3], Wf1, bf1, bcol)
    return _head2(g, Wa, ba, Wb, bb, Wc, bc)


# speedup vs baseline: 9.9952x; 1.0002x over previous
import functools

import jax
import jax.numpy as jnp
from jax import lax
from jax.experimental import pallas as pl
from jax.experimental.pallas import tpu as pltpu
from jax.experimental.pallas import tpu_sc as plsc

N = 4096
B = 8
KNN = 20
BIG = 1e30


def _lrelu(v):
    return jnp.where(v >= 0, v, 0.2 * v)


KP = 24


WIN = 2048


def _knn_body(win_ref, xb_ref, xT_ref, bcol_ref, brow_ref, idx_ref):
    i = pl.program_id(0)
    start = pl.multiple_of(win_ref[i, 0], 128)
    wide = win_ref[i, 1]
    xb = xb_ref[...]
    d2b = jnp.sum(xb * xb, axis=1, keepdims=True)
    bcol = bcol_ref[...]
    R = xb.shape[0]
    base = pl.program_id(0) * R

    def run(colstart, W_):
        xT = xT_ref[:, pl.ds(colstart, W_)]
        d2r = jnp.sum(xT * xT, axis=0, keepdims=True)
        xx = jnp.dot(xb, xT, preferred_element_type=jnp.float32)
        D = d2b + d2r - 2.0 * xx
        mask = bcol != brow_ref[:, pl.ds(colstart, W_)]
        D = jnp.where(mask, BIG, D)
        iota = colstart + jax.lax.broadcasted_iota(jnp.int32, (R, W_), 1)
        kiota = jax.lax.broadcasted_iota(jnp.int32, (R, KP), 1)
        riota = base + jax.lax.broadcasted_iota(jnp.int32, (R, KP), 0)

        def step(t, carry):
            D, IDX = carry
            m = jnp.min(D, axis=1, keepdims=True)
            am = jnp.min(jnp.where(D == m, iota, N), axis=1, keepdims=True)
            IDX = jnp.where(kiota == t, am, IDX)
            return jnp.where(iota == am, BIG, D), IDX

        _, IDX = jax.lax.fori_loop(0, KNN, step, (D, riota))
        idx_ref[...] = IDX

    @pl.when(wide == 0)
    def _():
        run(start, WIN)

    @pl.when(wide != 0)
    def _():
        run(0, N)


def _knn(x, xT, win, bcol, brow, R=256):
    d = x.shape[1]
    return pl.pallas_call(
        _knn_body,
        grid=(N // R,),
        in_specs=[
            pl.BlockSpec(memory_space=pltpu.SMEM),
            pl.BlockSpec((R, d), lambda i: (i, 0)),
            pl.BlockSpec((d, N), lambda i: (0, 0)),
            pl.BlockSpec((R, 1), lambda i: (i, 0)),
            pl.BlockSpec((1, N), lambda i: (0, 0)),
        ],
        out_specs=pl.BlockSpec((R, KP), lambda i: (i, 0)),
        out_shape=jax.ShapeDtypeStruct((N, KP), jnp.int32),
        compiler_params=pltpu.CompilerParams(
            dimension_semantics=("parallel",)),
    )(win, x, xT, bcol, brow)


def _windows(batch, R=256):
    firsts = batch[::R]
    lasts = batch[R - 1::R]
    seg_lo = jnp.searchsorted(batch, firsts, side="left").astype(jnp.int32)
    seg_hi = jnp.searchsorted(batch, lasts, side="right").astype(jnp.int32)
    sa = jnp.minimum(seg_lo, N - WIN) & ~jnp.int32(127)
    wide = (seg_hi > sa + WIN).astype(jnp.int32)
    return jnp.stack([sa, wide], axis=1)


_NC = 2
_NS = 16
_NW = _NC * _NS
_C = N // _NW


def _sc_gather(x, idxT):
    d = x.shape[1]
    mesh = plsc.VectorSubcoreMesh(core_axis_name="c", subcore_axis_name="s")

    @functools.partial(
        pl.kernel, mesh=mesh,
        out_type=jax.ShapeDtypeStruct((KNN, N, d), jnp.float32),
        scratch_types=(
            [pltpu.VMEM((_C,), jnp.int32) for _ in range(2)]
            + [pltpu.VMEM((_C, d), jnp.float32) for _ in range(3)]
            + [pltpu.SemaphoreType.DMA for _ in range(6)]
        ),
    )
    def k(x_hbm, idxT_hbm, xg_hbm, *scr):
        idxvs = scr[0:2]
        bufs = scr[2:5]
        gsems = scr[5:8]
        wsems = scr[8:11]
        wid = lax.axis_index("s") * _NC + lax.axis_index("c")
        base = wid * _C
        gcps = [None] * KNN
        wcps = [None] * KNN
        for k_ in range(KNN):
            if k_ >= 3:
                wcps[k_ - 3].wait()
            pltpu.sync_copy(idxT_hbm.at[k_, pl.ds(base, _C)], idxvs[k_ % 2])
            gcps[k_] = pltpu.async_copy(x_hbm.at[idxvs[k_ % 2]],
                                        bufs[k_ % 3], gsems[k_ % 3])
            if k_ > 0:
                gcps[k_ - 1].wait()
                wcps[k_ - 1] = pltpu.async_copy(
                    bufs[(k_ - 1) % 3],
                    xg_hbm.at[k_ - 1, pl.ds(base, _C), :],
                    wsems[(k_ - 1) % 3])
        gcps[KNN - 1].wait()
        wcps[KNN - 1] = pltpu.async_copy(
            bufs[(KNN - 1) % 3],
            xg_hbm.at[KNN - 1, pl.ds(base, _C), :],
            wsems[(KNN - 1) % 3])
        for k_ in range(KNN - 3, KNN):
            wcps[k_].wait()

    return k(x, idxT)


def _edge_body(d, xb_ref, xg_ref, w_ref, b_ref, o_ref):
    xb = xb_ref[...][:, :d]
    wbf = w_ref[...].astype(jnp.bfloat16)
    R = xb.shape[0]
    out = wbf.shape[1]
    M = jnp.full((R, out), -BIG, jnp.float32)
    for k_ in range(KNN):
        xj = xg_ref[k_][:, :d]
        msg = jnp.concatenate([xb, xj - xb], axis=1).astype(jnp.bfloat16)
        h = jnp.dot(msg, wbf, preferred_element_type=jnp.float32)
        M = jnp.maximum(M, h)
    o_ref[...] = _lrelu(M + b_ref[...])


def _edge_mlp(x, xg, W, b, d, R=512):
    dp = x.shape[1]
    out = W.shape[1]
    return pl.pallas_call(
        functools.partial(_edge_body, d),
        grid=(N // R,),
        in_specs=[
            pl.BlockSpec((R, dp), lambda i: (i, 0)),
            pl.BlockSpec((KNN, R, dp), lambda i: (0, i, 0)),
            pl.BlockSpec((2 * d, out), lambda i: (0, 0)),
            pl.BlockSpec((1, out), lambda i: (0, 0)),
        ],
        out_specs=pl.BlockSpec((R, out), lambda i: (i, 0)),
        out_shape=jax.ShapeDtypeStruct((N, out), jnp.float32),
        compiler_params=pltpu.CompilerParams(
            dimension_semantics=("parallel",)),
    )(x, xg, W, b.reshape(1, out))


def _edge_conv(x, win, bcol, brow, W, b):
    d = x.shape[1]
    dp = -(-d // 128) * 128
    xp = jnp.pad(x, ((0, 0), (0, dp - d))) if dp != d else x
    idx = _knn(x, x.T, win, bcol, brow)
    idxT = idx.T[:KNN]
    xg = _sc_gather(xp, idxT)
    return _edge_mlp(xp, xg, W, b, d)


def _head_body(x1_ref, x2_ref, x3_ref, x4_ref, wf_ref, bf_ref, bcol_ref, g_ref):
    h = jnp.concatenate(
        [x1_ref[...], x2_ref[...], x3_ref[...], x4_ref[...]], axis=1
    )
    h = _lrelu(jnp.dot(h, wf_ref[...], preferred_element_type=jnp.float32) + bf_ref[...])

    @pl.when(pl.program_id(0) == 0)
    def _():
        g_ref[...] = jnp.full(g_ref.shape, -jnp.inf, jnp.float32)

    bcol = bcol_ref[...]
    for seg in range(B):
        v = jnp.max(jnp.where(bcol == seg, h, -jnp.inf), axis=0, keepdims=True)
        g_ref[seg:seg + 1, :] = jnp.maximum(g_ref[seg:seg + 1, :], v)


def _head(x1, x2, x3, x4, Wf1, bf1, bcol, S=512):
    F = Wf1.shape[1]
    return pl.pallas_call(
        _head_body,
        grid=(N // S,),
        in_specs=[
            pl.BlockSpec((S, x1.shape[1]), lambda i: (i, 0)),
            pl.BlockSpec((S, x2.shape[1]), lambda i: (i, 0)),
            pl.BlockSpec((S, x3.shape[1]), lambda i: (i, 0)),
            pl.BlockSpec((S, x4.shape[1]), lambda i: (i, 0)),
            pl.BlockSpec(Wf1.shape, lambda i: (0, 0)),
            pl.BlockSpec((1, F), lambda i: (0, 0)),
            pl.BlockSpec((S, 1), lambda i: (i, 0)),
        ],
        out_specs=pl.BlockSpec((B, F), lambda i: (0, 0)),
        out_shape=jax.ShapeDtypeStruct((B, F), jnp.float32),
    )(x1, x2, x3, x4, Wf1, bf1.reshape(1, F), bcol)


def _head2_body(g_ref, wa_ref, ba_ref, wb_ref, bb_ref, wc_ref, bc_ref, o_ref):
    g = jnp.maximum(jnp.dot(g_ref[...], wa_ref[...], preferred_element_type=jnp.float32) + ba_ref[...], 0.0)
    g = jnp.maximum(jnp.dot(g, wb_ref[...], preferred_element_type=jnp.float32) + bb_ref[...], 0.0)
    z = jnp.dot(g, wc_ref[...], preferred_element_type=jnp.float32) + bc_ref[...]
    zmax = jnp.max(z, axis=1, keepdims=True)
    s = jnp.sum(jnp.exp(z - zmax), axis=1, keepdims=True)
    o_ref[...] = z - zmax - jnp.log(s)


def _head2(g, Wa, ba, Wb, bb, Wc, bc):
    out = Wc.shape[1]
    return pl.pallas_call(
        _head2_body,
        out_shape=jax.ShapeDtypeStruct((B, out), jnp.float32),
    )(g, Wa, ba.reshape(1, -1), Wb, bb.reshape(1, -1), Wc, bc.reshape(1, -1))


def kernel(pos, x, features, batch, W1, b1, W2, b2, W3, b3, W4, b4,
           Wf1, bf1, Wa, ba, Wb, bb, Wc, bc):
    bcol = batch.astype(jnp.int32).reshape(N, 1)
    brow = bcol.reshape(1, N)
    win = _windows(batch.astype(jnp.int32))
    h = jnp.concatenate([pos, x, features], axis=1)
    outs = []
    for (W, b) in ((W1, b1), (W2, b2), (W3, b3), (W4, b4)):
        h = _edge_conv(h, win, bcol, brow, W, b)
        outs.append(h)
    g = _head(outs[0], outs[1], outs[2], outs[3], Wf1, bf1, bcol)
    return _head2(g, Wa, ba, Wb, bb, Wc, bc)
